# baseline (speedup 1.0000x reference)
import functools

import jax
import jax.numpy as jnp
from jax import lax
from jax.experimental import pallas as pl

B, S, D_IN, D_HID, D_OUT = 4, 2048, 1024, 1024, 1024
N_PIECES = 20
S_BLK = 256
C_BLK = 128


def _phi_kernel(x_ref, w1_ref, b1_ref, w2_ref, b2_ref, o_ref):
    xb = x_ref[0]
    hm = jnp.dot(xb, w1_ref[...], preferred_element_type=jnp.float32)
    hm = jnp.maximum(hm + b1_ref[...], 0.0)
    h = jnp.dot(hm, w2_ref[...], preferred_element_type=jnp.float32)
    o_ref[0] = h + b2_ref[...]


def _bitonic_sort_asc(g, n, c):
    log_n = n.bit_length() - 1
    for ks in range(1, log_n + 1):
        size = 1 << ks
        for js in range(ks - 1, -1, -1):
            d = 1 << js
            m = n // (2 * d)
            x4 = g.reshape(m, 2, d, c)
            a = x4[:, 0]
            b = x4[:, 1]
            mn = jnp.minimum(a, b)
            mx = jnp.maximum(a, b)
            if size == n:
                first, second = mn, mx
            else:
                idx = lax.broadcasted_iota(jnp.int32, (m, 1, 1), 0)
                asc = (idx * (2 * d) & size) == 0
                first = jnp.where(asc, mn, mx)
                second = jnp.where(asc, mx, mn)
            g = jnp.concatenate([first[:, None], second[:, None]], axis=1
                                ).reshape(n, c)
    return g


def _sortpool_kernel(h_ref, wt_ref, o_ref):
    g = -h_ref[0]
    g = _bitonic_sort_asc(g, S, C_BLK)
    o_ref[0, 0, :] = -jnp.sum(g * wt_ref[...], axis=0)


def _rho_kernel(p_ref, w1_ref, b1_ref, w2_ref, b2_ref, o_ref):
    r = jnp.dot(p_ref[...], w1_ref[...], preferred_element_type=jnp.float32)
    r = jnp.maximum(r + b1_ref[...], 0.0)
    o = jnp.dot(r, w2_ref[...], preferred_element_type=jnp.float32)
    o_ref[...] = o + b2_ref[...]


@functools.partial(jax.jit, static_argnames=())
def kernel(x, phi_w1, phi_b1, phi_w2, phi_b2, pool_weight,
           rho_w1, rho_b1, rho_w2, rho_b2):
    f32 = jnp.float32

    h = pl.pallas_call(
        _phi_kernel,
        grid=(B, S // S_BLK),
        in_specs=[
            pl.BlockSpec((1, S_BLK, D_IN), lambda b, s: (b, s, 0)),
            pl.BlockSpec((D_IN, D_HID), lambda b, s: (0, 0)),
            pl.BlockSpec((1, D_HID), lambda b, s: (0, 0)),
            pl.BlockSpec((D_HID, D_HID), lambda b, s: (0, 0)),
            pl.BlockSpec((1, D_HID), lambda b, s: (0, 0)),
        ],
        out_specs=pl.BlockSpec((1, S_BLK, D_HID), lambda b, s: (b, s, 0)),
        out_shape=jax.ShapeDtypeStruct((B, S, D_HID), f32),
    )(x, phi_w1, phi_b1.reshape(1, -1), phi_w2, phi_b2.reshape(1, -1))

    pos = jnp.arange(S, dtype=f32) / jnp.maximum(1.0, float(S - 1))
    index = N_PIECES * pos
    idx = index.astype(jnp.int32)
    frac = index - idx.astype(f32)
    left = jnp.take(pool_weight, idx, axis=1)
    right = jnp.take(pool_weight, jnp.minimum(idx + 1, N_PIECES), axis=1)
    w = (1.0 - frac)[None, :] * left + frac[None, :] * right
    wt = w.T

    pooled = pl.pallas_call(
        _sortpool_kernel,
        grid=(B, D_HID // C_BLK),
        in_specs=[
            pl.BlockSpec((1, S, C_BLK), lambda b, c: (b, 0, c)),
            pl.BlockSpec((S, C_BLK), lambda b, c: (0, c)),
        ],
        out_specs=pl.BlockSpec((1, 1, C_BLK), lambda b, c: (b, 0, c)),
        out_shape=jax.ShapeDtypeStruct((B, 1, D_HID), f32),
    )(h, wt)
    pooled = pooled.reshape(B, D_HID)

    out = pl.pallas_call(
        _rho_kernel,
        in_specs=[
            pl.BlockSpec((B, D_HID), lambda: (0, 0)),
            pl.BlockSpec((D_HID, D_HID), lambda: (0, 0)),
            pl.BlockSpec((1, D_HID), lambda: (0, 0)),
            pl.BlockSpec((D_HID, D_OUT), lambda: (0, 0)),
            pl.BlockSpec((1, D_OUT), lambda: (0, 0)),
        ],
        out_specs=pl.BlockSpec((B, D_OUT), lambda: (0, 0)),
        out_shape=jax.ShapeDtypeStruct((B, D_OUT), f32),
    )(pooled, rho_w1, rho_b1.reshape(1, -1), rho_w2, rho_b2.reshape(1, -1))

    return out


# speedup vs baseline: 3.1079x; 3.1079x over previous
import functools

import jax
import jax.numpy as jnp
from jax import lax
from jax.experimental import pallas as pl

B, S, D_IN, D_HID, D_OUT = 4, 2048, 1024, 1024, 1024
N_PIECES = 20
S_BLK = 256
C_BLK = 128


def _phi_kernel(x_ref, w1_ref, b1_ref, w2_ref, b2_ref, o_ref):
    xb = x_ref[0]
    hm = jnp.dot(xb, w1_ref[...], preferred_element_type=jnp.float32)
    hm = jnp.maximum(hm + b1_ref[...], 0.0)
    h = jnp.dot(hm, w2_ref[...], preferred_element_type=jnp.float32)
    o_ref[0] = h + b2_ref[...]


def _ce_aligned(g, n, c, d):
    m = n // (2 * d)
    x4 = g.reshape(m, 2, d, c)
    mn = jnp.minimum(x4[:, 0], x4[:, 1])
    mx = jnp.maximum(x4[:, 0], x4[:, 1])
    return jnp.concatenate([mn[:, None], mx[:, None]], axis=1).reshape(n, c)


def _ce_small(g, ri, d):
    down = jnp.roll(g, -d, axis=0)
    up = jnp.roll(g, d, axis=0)
    is_first = (ri & d) == 0
    partner = jnp.where(is_first, down, up)
    mn = jnp.minimum(g, partner)
    mx = jnp.maximum(g, partner)
    return jnp.where(is_first, mn, mx)


def _bitonic_sort_asc(g, n, c):
    log_n = n.bit_length() - 1
    ri = lax.broadcasted_iota(jnp.int32, (n, 1), 0)
    for ks in range(1, log_n + 1):
        size = 1 << ks
        sign = None
        if size < n:
            sign = jnp.where((ri & size) == 0, 1.0, -1.0).astype(g.dtype)
            g = g * sign
        for js in range(ks - 1, -1, -1):
            d = 1 << js
            if d >= 8:
                g = _ce_aligned(g, n, c, d)
            else:
                g = _ce_small(g, ri, d)
        if sign is not None:
            g = g * sign
    return g


def _sortpool_kernel(h_ref, wt_ref, o_ref):
    g = -h_ref[0]
    g = _bitonic_sort_asc(g, S, C_BLK)
    o_ref[0, 0, :] = -jnp.sum(g * wt_ref[...], axis=0)


def _rho_kernel(p_ref, w1_ref, b1_ref, w2_ref, b2_ref, o_ref):
    r = jnp.dot(p_ref[...], w1_ref[...], preferred_element_type=jnp.float32)
    r = jnp.maximum(r + b1_ref[...], 0.0)
    o = jnp.dot(r, w2_ref[...], preferred_element_type=jnp.float32)
    o_ref[...] = o + b2_ref[...]


@functools.partial(jax.jit, static_argnames=())
def kernel(x, phi_w1, phi_b1, phi_w2, phi_b2, pool_weight,
           rho_w1, rho_b1, rho_w2, rho_b2):
    f32 = jnp.float32

    h = pl.pallas_call(
        _phi_kernel,
        grid=(B, S // S_BLK),
        in_specs=[
            pl.BlockSpec((1, S_BLK, D_IN), lambda b, s: (b, s, 0)),
            pl.BlockSpec((D_IN, D_HID), lambda b, s: (0, 0)),
            pl.BlockSpec((1, D_HID), lambda b, s: (0, 0)),
            pl.BlockSpec((D_HID, D_HID), lambda b, s: (0, 0)),
            pl.BlockSpec((1, D_HID), lambda b, s: (0, 0)),
        ],
        out_specs=pl.BlockSpec((1, S_BLK, D_HID), lambda b, s: (b, s, 0)),
        out_shape=jax.ShapeDtypeStruct((B, S, D_HID), f32),
    )(x, phi_w1, phi_b1.reshape(1, -1), phi_w2, phi_b2.reshape(1, -1))

    pos = jnp.arange(S, dtype=f32) / jnp.maximum(1.0, float(S - 1))
    index = N_PIECES * pos
    idx = index.astype(jnp.int32)
    frac = index - idx.astype(f32)
    left = jnp.take(pool_weight, idx, axis=1)
    right = jnp.take(pool_weight, jnp.minimum(idx + 1, N_PIECES), axis=1)
    w = (1.0 - frac)[None, :] * left + frac[None, :] * right
    wt = w.T

    pooled = pl.pallas_call(
        _sortpool_kernel,
        grid=(B, D_HID // C_BLK),
        in_specs=[
            pl.BlockSpec((1, S, C_BLK), lambda b, c: (b, 0, c)),
            pl.BlockSpec((S, C_BLK), lambda b, c: (0, c)),
        ],
        out_specs=pl.BlockSpec((1, 1, C_BLK), lambda b, c: (b, 0, c)),
        out_shape=jax.ShapeDtypeStruct((B, 1, D_HID), f32),
    )(h, wt)
    pooled = pooled.reshape(B, D_HID)

    out = pl.pallas_call(
        _rho_kernel,
        in_specs=[
            pl.BlockSpec((B, D_HID), lambda: (0, 0)),
            pl.BlockSpec((D_HID, D_HID), lambda: (0, 0)),
            pl.BlockSpec((1, D_HID), lambda: (0, 0)),
            pl.BlockSpec((D_HID, D_OUT), lambda: (0, 0)),
            pl.BlockSpec((1, D_OUT), lambda: (0, 0)),
        ],
        out_specs=pl.BlockSpec((B, D_OUT), lambda: (0, 0)),
        out_shape=jax.ShapeDtypeStruct((B, D_OUT), f32),
    )(pooled, rho_w1, rho_b1.reshape(1, -1), rho_w2, rho_b2.reshape(1, -1))

    return out
